# baseline (speedup 1.0000x reference)
import functools

import jax
import jax.numpy as jnp
from jax import lax
from jax.experimental import pallas as pl
from jax.experimental.pallas import tpu as pltpu
from jax.experimental.pallas import tpu_sc as plsc

N = 16777216
S = 2097152
M = N - S

NC = 2
NS = 16
NW = NC * NS
L = 16

C = 16384
UNROLL = 8

TC_BLK = 1048576


def _compute_chunk(xb, yb, ob, b):
    @plsc.parallel_loop(0, C, step=L, unroll=UNROLL)
    def body(off):
        yv = yb[b, pl.ds(off, L)]
        xv = xb[b, pl.ds(off, L)]
        nz = yv != 0.0
        inv = jnp.where(nz, 1.0 / jnp.where(nz, yv, 1.0), 0.0)
        ob[b, pl.ds(off, L)] = inv * xv


def _sc_body(x_hbm, y_hbm, o_hbm, xb, yb, ob, insem0, insem1, outsem0,
             outsem1):
    wid = lax.axis_index("s") * NC + lax.axis_index("c")
    per_w = S // NW
    T = per_w // C
    G = T // 2
    base_in = M + wid * per_w
    base_out = wid * per_w

    def start_in(t, b, sem):
        pltpu.async_copy(x_hbm.at[pl.ds(base_in + t * C, C)], xb.at[b], sem)
        pltpu.async_copy(y_hbm.at[pl.ds(base_in + t * C, C)], yb.at[b], sem)

    def wait_in(b, sem):
        pltpu.make_async_copy(x_hbm.at[pl.ds(0, C)], xb.at[b], sem).wait()
        pltpu.make_async_copy(y_hbm.at[pl.ds(0, C)], yb.at[b], sem).wait()

    def start_out(t, b, sem):
        pltpu.async_copy(ob.at[b], o_hbm.at[pl.ds(base_out + t * C, C)], sem)

    def wait_out(b, sem):
        pltpu.make_async_copy(ob.at[b], o_hbm.at[pl.ds(0, C)], sem).wait()

    start_in(0, 0, insem0)

    def ring(g, _):
        t0 = 2 * g
        start_in(t0 + 1, 1, insem1)
        wait_in(0, insem0)

        @pl.when(g >= 1)
        def _():
            wait_out(0, outsem0)

        _compute_chunk(xb, yb, ob, 0)
        start_out(t0, 0, outsem0)

        @pl.when(g < G - 1)
        def _():
            start_in(t0 + 2, 0, insem0)

        wait_in(1, insem1)

        @pl.when(g >= 1)
        def _():
            wait_out(1, outsem1)

        _compute_chunk(xb, yb, ob, 1)
        start_out(t0 + 1, 1, outsem1)
        return 0

    lax.fori_loop(0, G, ring, 0)
    wait_out(0, outsem0)
    wait_out(1, outsem1)


@functools.partial(
    pl.kernel,
    mesh=plsc.VectorSubcoreMesh(core_axis_name="c", subcore_axis_name="s"),
    out_type=jax.ShapeDtypeStruct((S,), jnp.float32),
    scratch_types=[
        pltpu.VMEM((2, C), jnp.float32),
        pltpu.VMEM((2, C), jnp.float32),
        pltpu.VMEM((2, C), jnp.float32),
        pltpu.SemaphoreType.DMA,
        pltpu.SemaphoreType.DMA,
        pltpu.SemaphoreType.DMA,
        pltpu.SemaphoreType.DMA,
    ],
)
def _sc_tail(x_hbm, y_hbm, o_hbm, xb, yb, ob, insem0, insem1, outsem0,
             outsem1):
    _sc_body(x_hbm, y_hbm, o_hbm, xb, yb, ob, insem0, insem1, outsem0,
             outsem1)


def _tc_body(x_ref, y_ref, o_ref):
    yv = y_ref[...]
    xv = x_ref[...]
    nz = yv != 0.0
    inv = jnp.where(nz, 1.0 / jnp.where(nz, yv, 1.0), 0.0)
    o_ref[...] = inv * xv


def kernel(x, y):
    tc_out = pl.pallas_call(
        _tc_body,
        grid=(M // TC_BLK,),
        in_specs=[
            pl.BlockSpec((TC_BLK,), lambda i: (i,)),
            pl.BlockSpec((TC_BLK,), lambda i: (i,)),
        ],
        out_specs=pl.BlockSpec((TC_BLK,), lambda i: (i,)),
        out_shape=jax.ShapeDtypeStruct((N,), jnp.float32),
    )(x, y)
    sc_out = _sc_tail(x, y)
    return lax.dynamic_update_slice(tc_out, sc_out, (M,))


# speedup vs baseline: 1.3587x; 1.3587x over previous
import jax
import jax.numpy as jnp
from jax.experimental import pallas as pl


def _body(x_ref, y_ref, o_ref):
    yv = y_ref[...]
    xv = x_ref[...]
    nz = yv != 0.0
    inv = jnp.where(nz, 1.0 / jnp.where(nz, yv, 1.0), 0.0)
    o_ref[...] = inv * xv


def kernel(x, y):
    N = x.shape[0]
    BLK = 1048576
    out = pl.pallas_call(
        _body,
        grid=(N // BLK,),
        in_specs=[
            pl.BlockSpec((BLK,), lambda i: (i,)),
            pl.BlockSpec((BLK,), lambda i: (i,)),
        ],
        out_specs=pl.BlockSpec((BLK,), lambda i: (i,)),
        out_shape=jax.ShapeDtypeStruct((N,), jnp.float32),
    )(x, y)
    return out


# speedup vs baseline: 1.3593x; 1.0005x over previous
import jax
import jax.numpy as jnp
from jax.experimental import pallas as pl
from jax.experimental.pallas import tpu as pltpu


def _body(x_ref, y_ref, o_ref):
    yv = y_ref[...]
    xv = x_ref[...]
    nz = yv != 0.0
    inv = jnp.where(nz, 1.0 / jnp.where(nz, yv, 1.0), 0.0)
    o_ref[...] = inv * xv


def kernel(x, y):
    N = x.shape[0]
    BLK = 1048576
    out = pl.pallas_call(
        _body,
        grid=(N // BLK,),
        in_specs=[
            pl.BlockSpec((BLK,), lambda i: (i,)),
            pl.BlockSpec((BLK,), lambda i: (i,)),
        ],
        out_specs=pl.BlockSpec((BLK,), lambda i: (i,)),
        out_shape=jax.ShapeDtypeStruct((N,), jnp.float32),
        compiler_params=pltpu.CompilerParams(
            dimension_semantics=("arbitrary",),
            skip_device_barrier=True,
        ),
    )(x, y)
    return out


# speedup vs baseline: 1.3617x; 1.0018x over previous
import jax
import jax.numpy as jnp
from jax.experimental import pallas as pl


def _body(x_ref, y_ref, o_ref):
    yv = y_ref[...]
    xv = x_ref[...]
    inv = 1.0 / yv
    o_ref[...] = jnp.where(yv != 0.0, inv * xv, 0.0)


def kernel(x, y):
    N = x.shape[0]
    BLK = 1048576
    out = pl.pallas_call(
        _body,
        grid=(N // BLK,),
        in_specs=[
            pl.BlockSpec((BLK,), lambda i: (i,)),
            pl.BlockSpec((BLK,), lambda i: (i,)),
        ],
        out_specs=pl.BlockSpec((BLK,), lambda i: (i,)),
        out_shape=jax.ShapeDtypeStruct((N,), jnp.float32),
    )(x, y)
    return out
